# baseline (device time: 25247 ns/iter reference)
import os

import jax
import jax.numpy as jnp
from jax import lax
from jax.experimental import pallas as pl
from jax.experimental.pallas import tpu as pltpu

N_DEV = 8
B, SQ, SKV, H_LOC, DH = 2, 128, 128, 4, 64
D_MODEL = 512
ROUNDS = (1, 3, 4)
HALF_ROUNDS = ((1, 3, 4), (4, 1, 3))
NCB = 4
CH = B * NCB
CROWS = 128
CCOLS = D_MODEL // NCB

_PROBE_NO_COMM = os.environ.get("PROBE_NO_COMM") == "1"


def kernel(x, Wq, K_ext, V_ext, Wo):
    my = lax.axis_index("i")
    c0 = my * (H_LOC * DH)
    K2 = lax.dynamic_slice(
        K_ext.reshape(B * SKV, 32 * DH), (0, c0), (B * SKV, H_LOC * DH)
    )
    V2 = lax.dynamic_slice(
        V_ext.reshape(B * SKV, 32 * DH), (0, c0), (B * SKV, H_LOC * DH)
    )
    x2 = x.reshape(B * SQ, D_MODEL)

    def body(x_ref, wq_ref, k_ref, v_ref, wo_ref, out_ref,
             send_ref, recv_ref, send_sems, recv_sems):
        my_pos = lax.axis_index("i")

        barrier_sem = pltpu.get_barrier_semaphore()
        partners = [my_pos ^ m for m in ROUNDS]
        for p in partners:
            pl.semaphore_signal(
                barrier_sem, inc=1,
                device_id=(p,), device_id_type=pl.DeviceIdType.MESH,
            )

        def mk(r, j):
            mask = HALF_ROUNDS[j % 2][r]
            return pltpu.make_async_remote_copy(
                src_ref=send_ref.at[j],
                dst_ref=recv_ref.at[r, j],
                send_sem=send_sems.at[r, j],
                recv_sem=recv_sems.at[r, j],
                device_id=(my_pos ^ mask,),
                device_id_type=pl.DeviceIdType.MESH,
            )

        wq = wq_ref[:].astype(jnp.bfloat16)
        wo = wo_ref[:].astype(jnp.bfloat16)
        k2 = k_ref[:].astype(jnp.bfloat16)
        v2 = v_ref[:].astype(jnp.bfloat16)

        xb = x_ref[:].astype(jnp.bfloat16)
        q_all = lax.dot(xb, wq, preferred_element_type=jnp.float32)
        q_all = (q_all * 0.125).astype(jnp.bfloat16)

        def attention_batch(b):
            q = q_all[b * SQ:(b + 1) * SQ, :]
            head_ctx = []
            for h in range(H_LOC):
                qb = q[:, h * DH:(h + 1) * DH]
                kb = k2[b * SKV:(b + 1) * SKV, h * DH:(h + 1) * DH]
                vb = v2[b * SKV:(b + 1) * SKV, h * DH:(h + 1) * DH]
                s = lax.dot_general(
                    qb, kb, (((1,), (1,)), ((), ())),
                    preferred_element_type=jnp.float32,
                )
                w = jnp.exp(s)
                rs = 1.0 / jnp.sum(w, axis=-1, keepdims=True)
                cx = lax.dot(w.astype(jnp.bfloat16), vb,
                             preferred_element_type=jnp.float32)
                head_ctx.append(cx * rs)
            ctx = jnp.concatenate(head_ctx, axis=1)
            return ctx.astype(jnp.bfloat16)

        accs = [None] * CH
        rdmas = {}
        for b in range(B):
            ctx = attention_batch(b)
            for cb in range(NCB):
                j = b * NCB + cb
                accs[j] = lax.dot(
                    ctx, wo[:, cb * CCOLS:(cb + 1) * CCOLS],
                    preferred_element_type=jnp.float32,
                ).astype(jnp.bfloat16)
                if not _PROBE_NO_COMM:
                    send_ref[j] = accs[j]
                    if j == 0:
                        pl.semaphore_wait(barrier_sem, len(partners))
                    d = mk(0, j)
                    d.start()
                    rdmas[(0, j)] = d

        def store(j, val):
            out_ref[pl.ds((j // NCB) * CROWS, CROWS),
                    pl.ds((j % NCB) * CCOLS, CCOLS)] = val

        if _PROBE_NO_COMM:
            for j in range(CH):
                store(j, accs[j])
            return

        for r in range(len(ROUNDS)):
            for j in range(CH):
                rdmas[(r, j)].wait()
                accs[j] = accs[j] + recv_ref[r, j]
                if r < len(ROUNDS) - 1:
                    send_ref[j] = accs[j]
                    d = mk(r + 1, j)
                    d.start()
                    rdmas[(r + 1, j)] = d
                else:
                    store(j, accs[j])


    out = pl.pallas_call(
        body,
        out_shape=jax.ShapeDtypeStruct((B * SQ, D_MODEL), jnp.bfloat16),
        in_specs=[pl.BlockSpec(memory_space=pltpu.VMEM)] * 5,
        out_specs=pl.BlockSpec(memory_space=pltpu.VMEM),
        scratch_shapes=[
            pltpu.VMEM((CH, CROWS, CCOLS), jnp.bfloat16),
            pltpu.VMEM((3, CH, CROWS, CCOLS), jnp.bfloat16),
            pltpu.SemaphoreType.DMA((3, CH)),
            pltpu.SemaphoreType.DMA((3, CH)),
        ],
        compiler_params=pltpu.CompilerParams(collective_id=0),
    )(x2, Wq, K2, V2, Wo)
    return out.reshape(B, SQ, D_MODEL)


# device time: 22778 ns/iter; 1.1084x vs baseline; 1.1084x over previous
import os

import jax
import jax.numpy as jnp
from jax import lax
from jax.experimental import pallas as pl
from jax.experimental.pallas import tpu as pltpu

N_DEV = 8
B, SQ, SKV, H_LOC, DH = 2, 128, 128, 4, 64
D_MODEL = 512
ROUNDS = (1, 3, 4)
HALF_ROUNDS = ((1, 3, 4), (4, 1, 3))
NCB = 4
CH = B * NCB
CROWS = 128
CCOLS = D_MODEL // NCB

_PROBE_NO_COMM = os.environ.get("PROBE_NO_COMM") == "1"


def kernel(x, Wq, K_ext, V_ext, Wo):
    def body(x_ref, wq_ref, k_hbm, v_hbm, wo_ref, out_ref,
             kv_ref, send_ref, recv_ref, kv_sems, send_sems, recv_sems):
        my_pos = lax.axis_index("i")
        h0 = my_pos * H_LOC

        kv_copies = []
        for t, hbm in enumerate((k_hbm, v_hbm)):
            c = pltpu.make_async_copy(
                hbm.at[:, :, pl.ds(h0, H_LOC), :],
                kv_ref.at[t],
                kv_sems.at[t],
            )
            c.start()
            kv_copies.append(c)

        barrier_sem = pltpu.get_barrier_semaphore()
        partners = [my_pos ^ m for m in ROUNDS]
        for p in partners:
            pl.semaphore_signal(
                barrier_sem, inc=1,
                device_id=(p,), device_id_type=pl.DeviceIdType.MESH,
            )

        def mk(r, j):
            mask = HALF_ROUNDS[j % 2][r]
            return pltpu.make_async_remote_copy(
                src_ref=send_ref.at[j],
                dst_ref=recv_ref.at[r, j],
                send_sem=send_sems.at[r, j],
                recv_sem=recv_sems.at[r, j],
                device_id=(my_pos ^ mask,),
                device_id_type=pl.DeviceIdType.MESH,
            )

        wq = wq_ref[:].astype(jnp.bfloat16)
        wo = wo_ref[:].astype(jnp.bfloat16)

        xb = x_ref[:].reshape(B * SQ, D_MODEL).astype(jnp.bfloat16)
        q_all = lax.dot(xb, wq, preferred_element_type=jnp.float32)
        q_all = (q_all * 0.125).astype(jnp.bfloat16)

        for c in kv_copies:
            c.wait()

        def attention_batch(b):
            q = q_all[b * SQ:(b + 1) * SQ, :]
            head_ctx = []
            for h in range(H_LOC):
                qb = q[:, h * DH:(h + 1) * DH]
                kb = kv_ref[0, b, :, h, :].astype(jnp.bfloat16)
                vb = kv_ref[1, b, :, h, :].astype(jnp.bfloat16)
                s = lax.dot_general(
                    qb, kb, (((1,), (1,)), ((), ())),
                    preferred_element_type=jnp.float32,
                )
                w = jnp.exp(s)
                rs = 1.0 / jnp.sum(w, axis=-1, keepdims=True)
                cx = lax.dot(w.astype(jnp.bfloat16), vb,
                             preferred_element_type=jnp.float32)
                head_ctx.append(cx * rs)
            ctx = jnp.concatenate(head_ctx, axis=1)
            return ctx.astype(jnp.bfloat16)

        accs = [None] * CH
        rdmas = {}
        for b in range(B):
            ctx = attention_batch(b)
            for cb in range(NCB):
                j = b * NCB + cb
                accs[j] = lax.dot(
                    ctx, wo[:, cb * CCOLS:(cb + 1) * CCOLS],
                    preferred_element_type=jnp.float32,
                ).astype(jnp.bfloat16)
                if not _PROBE_NO_COMM:
                    send_ref[j] = accs[j]
                    if j == 0:
                        pl.semaphore_wait(barrier_sem, len(partners))
                    d = mk(0, j)
                    d.start()
                    rdmas[(0, j)] = d

        def store(j, val):
            out_ref[j // NCB, :, pl.ds((j % NCB) * CCOLS, CCOLS)] = val

        if _PROBE_NO_COMM:
            for j in range(CH):
                store(j, accs[j])
            return

        for r in range(len(ROUNDS)):
            for j in range(CH):
                rdmas[(r, j)].wait()
                accs[j] = accs[j] + recv_ref[r, j]
                if r < len(ROUNDS) - 1:
                    send_ref[j] = accs[j]
                    d = mk(r + 1, j)
                    d.start()
                    rdmas[(r + 1, j)] = d
                else:
                    store(j, accs[j])


    return pl.pallas_call(
        body,
        out_shape=jax.ShapeDtypeStruct((B, SQ, D_MODEL), jnp.bfloat16),
        in_specs=[
            pl.BlockSpec(memory_space=pltpu.VMEM),
            pl.BlockSpec(memory_space=pltpu.VMEM),
            pl.BlockSpec(memory_space=pltpu.MemorySpace.HBM),
            pl.BlockSpec(memory_space=pltpu.MemorySpace.HBM),
            pl.BlockSpec(memory_space=pltpu.VMEM),
        ],
        out_specs=pl.BlockSpec(memory_space=pltpu.VMEM),
        scratch_shapes=[
            pltpu.VMEM((2, B, SKV, H_LOC, DH), jnp.float32),
            pltpu.VMEM((CH, CROWS, CCOLS), jnp.bfloat16),
            pltpu.VMEM((3, CH, CROWS, CCOLS), jnp.bfloat16),
            pltpu.SemaphoreType.DMA((2,)),
            pltpu.SemaphoreType.DMA((3, CH)),
            pltpu.SemaphoreType.DMA((3, CH)),
        ],
        compiler_params=pltpu.CompilerParams(collective_id=0),
    )(x, Wq, K_ext, V_ext, Wo)


# device time: 15573 ns/iter; 1.6212x vs baseline; 1.4627x over previous
import os

import jax
import jax.numpy as jnp
from jax import lax
from jax.experimental import pallas as pl
from jax.experimental.pallas import tpu as pltpu

N_DEV = 8
B, SQ, SKV, H_LOC, DH = 2, 128, 128, 4, 64
D_MODEL = 512
ROUNDS = (1, 3, 4)
HALF_ROUNDS = ((1, 3, 4), (4, 1, 3))
NCB = 4
CH = B * NCB
CROWS = 128
CCOLS = D_MODEL // NCB

_PROBE_NO_COMM = os.environ.get("PROBE_NO_COMM") == "1"


def kernel(x, Wq, K_ext, V_ext, Wo):
    my = lax.axis_index("i")
    h0 = my * H_LOC
    K2 = lax.dynamic_slice(
        K_ext, (0, 0, h0, 0), (B, SKV, H_LOC, DH)
    ).reshape(B * SKV, H_LOC * DH)
    V2 = lax.dynamic_slice(
        V_ext, (0, 0, h0, 0), (B, SKV, H_LOC, DH)
    ).reshape(B * SKV, H_LOC * DH)

    def body(x_ref, wq_ref, k_ref, v_ref, wo_ref, out_ref,
             send_ref, recv_ref, send_sems, recv_sems):
        my_pos = lax.axis_index("i")

        barrier_sem = pltpu.get_barrier_semaphore()
        partners = [my_pos ^ m for m in ROUNDS]
        for p in partners:
            pl.semaphore_signal(
                barrier_sem, inc=1,
                device_id=(p,), device_id_type=pl.DeviceIdType.MESH,
            )

        def mk(r, j):
            mask = HALF_ROUNDS[j % 2][r]
            return pltpu.make_async_remote_copy(
                src_ref=send_ref.at[j],
                dst_ref=recv_ref.at[r, j],
                send_sem=send_sems.at[r, j],
                recv_sem=recv_sems.at[r, j],
                device_id=(my_pos ^ mask,),
                device_id_type=pl.DeviceIdType.MESH,
            )

        wq = wq_ref[:].astype(jnp.bfloat16)
        wo = wo_ref[:].astype(jnp.bfloat16)
        k2 = k_ref[:].astype(jnp.bfloat16)
        v2 = v_ref[:].astype(jnp.bfloat16)

        xb = x_ref[:].reshape(B * SQ, D_MODEL).astype(jnp.bfloat16)
        q_all = lax.dot(xb, wq, preferred_element_type=jnp.float32)
        q_all = (q_all * 0.125).astype(jnp.bfloat16)

        def attention_batch(b):
            q = q_all[b * SQ:(b + 1) * SQ, :]
            head_ctx = []
            for h in range(H_LOC):
                qb = q[:, h * DH:(h + 1) * DH]
                kb = k2[b * SKV:(b + 1) * SKV, h * DH:(h + 1) * DH]
                vb = v2[b * SKV:(b + 1) * SKV, h * DH:(h + 1) * DH]
                s = lax.dot_general(
                    qb, kb, (((1,), (1,)), ((), ())),
                    preferred_element_type=jnp.float32,
                )
                w = jnp.exp(s)
                rs = 1.0 / jnp.sum(w, axis=-1, keepdims=True)
                cx = lax.dot(w.astype(jnp.bfloat16), vb,
                             preferred_element_type=jnp.float32)
                head_ctx.append(cx * rs)
            ctx = jnp.concatenate(head_ctx, axis=1)
            return ctx.astype(jnp.bfloat16)

        accs = [None] * CH
        rdmas = {}
        for b in range(B):
            ctx = attention_batch(b)
            for cb in range(NCB):
                j = b * NCB + cb
                accs[j] = lax.dot(
                    ctx, wo[:, cb * CCOLS:(cb + 1) * CCOLS],
                    preferred_element_type=jnp.float32,
                ).astype(jnp.bfloat16)
                if not _PROBE_NO_COMM:
                    send_ref[j] = accs[j]
                    if j == 0:
                        pl.semaphore_wait(barrier_sem, len(partners))
                    d = mk(0, j)
                    d.start()
                    rdmas[(0, j)] = d

        def store(j, val):
            out_ref[j // NCB, :, pl.ds((j % NCB) * CCOLS, CCOLS)] = val

        if _PROBE_NO_COMM:
            for j in range(CH):
                store(j, accs[j])
            return

        for r in range(len(ROUNDS)):
            for j in range(CH):
                rdmas[(r, j)].wait()
                accs[j] = accs[j] + recv_ref[r, j]
                if r < len(ROUNDS) - 1:
                    send_ref[j] = accs[j]
                    d = mk(r + 1, j)
                    d.start()
                    rdmas[(r + 1, j)] = d
                else:
                    store(j, accs[j])


    return pl.pallas_call(
        body,
        out_shape=jax.ShapeDtypeStruct((B, SQ, D_MODEL), jnp.bfloat16),
        in_specs=[pl.BlockSpec(memory_space=pltpu.VMEM)] * 5,
        out_specs=pl.BlockSpec(memory_space=pltpu.VMEM),
        scratch_shapes=[
            pltpu.VMEM((CH, CROWS, CCOLS), jnp.bfloat16),
            pltpu.VMEM((3, CH, CROWS, CCOLS), jnp.bfloat16),
            pltpu.SemaphoreType.DMA((3, CH)),
            pltpu.SemaphoreType.DMA((3, CH)),
        ],
        compiler_params=pltpu.CompilerParams(collective_id=0),
    )(x, Wq, K2, V2, Wo)


# device time: 15440 ns/iter; 1.6352x vs baseline; 1.0086x over previous
import os

import jax
import jax.numpy as jnp
from jax import lax
from jax.experimental import pallas as pl
from jax.experimental.pallas import tpu as pltpu

N_DEV = 8
B, SQ, SKV, H_LOC, DH = 2, 128, 128, 4, 64
D_MODEL = 512
ROUNDS = (1, 3, 4)
HALF_ROUNDS = ((1, 3, 4), (4, 1, 3))
NCB = 4
CH = B * NCB
CROWS = 128
CCOLS = D_MODEL // NCB

_PROBE_NO_COMM = os.environ.get("PROBE_NO_COMM") == "1"


def kernel(x, Wq, K_ext, V_ext, Wo):
    my = lax.axis_index("i")
    h0 = my * H_LOC
    K2 = lax.dynamic_slice(
        K_ext, (0, 0, h0, 0), (B, SKV, H_LOC, DH)
    ).reshape(B * SKV, H_LOC * DH).astype(jnp.bfloat16)
    V2 = lax.dynamic_slice(
        V_ext, (0, 0, h0, 0), (B, SKV, H_LOC, DH)
    ).reshape(B * SKV, H_LOC * DH).astype(jnp.bfloat16)
    xb16 = x.astype(jnp.bfloat16)

    def body(x_ref, wq_ref, k_ref, v_ref, wo_ref, out_ref,
             send_ref, recv_ref, send_sems, recv_sems):
        my_pos = lax.axis_index("i")

        barrier_sem = pltpu.get_barrier_semaphore()
        partners = [my_pos ^ m for m in ROUNDS]
        for p in partners:
            pl.semaphore_signal(
                barrier_sem, inc=1,
                device_id=(p,), device_id_type=pl.DeviceIdType.MESH,
            )

        def mk(r, j):
            mask = HALF_ROUNDS[j % 2][r]
            return pltpu.make_async_remote_copy(
                src_ref=send_ref.at[j],
                dst_ref=recv_ref.at[r, j],
                send_sem=send_sems.at[r, j],
                recv_sem=recv_sems.at[r, j],
                device_id=(my_pos ^ mask,),
                device_id_type=pl.DeviceIdType.MESH,
            )

        wq = wq_ref[:].astype(jnp.bfloat16)
        wo = wo_ref[:].astype(jnp.bfloat16)
        k2 = k_ref[:]
        v2 = v_ref[:]

        xb = x_ref[:].reshape(B * SQ, D_MODEL)
        q_all = lax.dot(xb, wq, preferred_element_type=jnp.float32)
        q_all = (q_all * 0.125).astype(jnp.bfloat16)

        def attention_batch(b):
            q = q_all[b * SQ:(b + 1) * SQ, :]
            head_ctx = []
            for h in range(H_LOC):
                qb = q[:, h * DH:(h + 1) * DH]
                kb = k2[b * SKV:(b + 1) * SKV, h * DH:(h + 1) * DH]
                vb = v2[b * SKV:(b + 1) * SKV, h * DH:(h + 1) * DH]
                s = lax.dot_general(
                    qb, kb, (((1,), (1,)), ((), ())),
                    preferred_element_type=jnp.float32,
                )
                w = jnp.exp(s)
                rs = 1.0 / jnp.sum(w, axis=-1, keepdims=True)
                cx = lax.dot(w.astype(jnp.bfloat16), vb,
                             preferred_element_type=jnp.float32)
                head_ctx.append(cx * rs)
            ctx = jnp.concatenate(head_ctx, axis=1)
            return ctx.astype(jnp.bfloat16)

        accs = [None] * CH
        rdmas = {}
        for b in range(B):
            ctx = attention_batch(b)
            for cb in range(NCB):
                j = b * NCB + cb
                accs[j] = lax.dot(
                    ctx, wo[:, cb * CCOLS:(cb + 1) * CCOLS],
                    preferred_element_type=jnp.float32,
                ).astype(jnp.bfloat16)
                if not _PROBE_NO_COMM:
                    send_ref[j] = accs[j]
                    if j == 0:
                        pl.semaphore_wait(barrier_sem, len(partners))
                    d = mk(0, j)
                    d.start()
                    rdmas[(0, j)] = d

        def store(j, val):
            out_ref[j // NCB, :, pl.ds((j % NCB) * CCOLS, CCOLS)] = val

        if _PROBE_NO_COMM:
            for j in range(CH):
                store(j, accs[j])
            return

        for r in range(len(ROUNDS)):
            for j in range(CH):
                rdmas[(r, j)].wait()
                accs[j] = accs[j] + recv_ref[r, j]
                if r < len(ROUNDS) - 1:
                    send_ref[j] = accs[j]
                    d = mk(r + 1, j)
                    d.start()
                    rdmas[(r + 1, j)] = d
                else:
                    store(j, accs[j])


    return pl.pallas_call(
        body,
        out_shape=jax.ShapeDtypeStruct((B, SQ, D_MODEL), jnp.bfloat16),
        in_specs=[pl.BlockSpec(memory_space=pltpu.VMEM)] * 5,
        out_specs=pl.BlockSpec(memory_space=pltpu.VMEM),
        scratch_shapes=[
            pltpu.VMEM((CH, CROWS, CCOLS), jnp.bfloat16),
            pltpu.VMEM((3, CH, CROWS, CCOLS), jnp.bfloat16),
            pltpu.SemaphoreType.DMA((3, CH)),
            pltpu.SemaphoreType.DMA((3, CH)),
        ],
        compiler_params=pltpu.CompilerParams(collective_id=0),
    )(xb16, Wq, K2, V2, Wo)
